# baseline (speedup 1.0000x reference)
import math

import jax
import jax.numpy as jnp
from jax.experimental import pallas as pl
from jax.experimental.pallas import tpu as pltpu

SPAT_DIM = 48
TIME_DIM = 48
MAX_N_TIMES = 30000
NUM_CHANNELS = 64

_CPT = 8


def _time_encoding(n_times: int) -> jnp.ndarray:
    position = jnp.arange(n_times, dtype=jnp.float32)[:, None]
    div_term = jnp.exp(
        jnp.arange(0, TIME_DIM, 2, dtype=jnp.float32)
        * (-math.log(MAX_N_TIMES) / TIME_DIM)
    )
    s = jnp.sin(position * div_term)
    c = jnp.cos(position * div_term)
    return jnp.stack([s, c], axis=-1).reshape(n_times, TIME_DIM)


def _encode_kernel(idx_ref, wt_ref, tt_ref, out_ref):
    b = pl.program_id(0)
    j = pl.program_id(1)
    n_times = tt_ref.shape[1] // _CPT
    out_ref[0, pl.ds(SPAT_DIM, TIME_DIM), :] = tt_ref[:, :]
    chan_iota = jax.lax.broadcasted_iota(jnp.int32, (NUM_CHANNELS, 1), 0)
    for k in range(_CPT):
        cidx = idx_ref[b, j * _CPT + k]
        onehot = (chan_iota == cidx).astype(jnp.float32)
        col = jax.lax.dot_general(
            wt_ref[:, :],
            onehot,
            (((1,), (0,)), ((), ())),
            preferred_element_type=jnp.float32,
        )
        out_ref[0, pl.ds(0, SPAT_DIM), pl.ds(k * n_times, n_times)] = (
            jnp.broadcast_to(col, (SPAT_DIM, n_times))
        )


def kernel(local_features, ch_idxs, W_spat):
    batch_size, n_chans_times, emb_dim = local_features.shape
    _, n_chans = ch_idxs.shape
    n_times = n_chans_times // n_chans
    t_enc = _time_encoding(n_times)
    wt = W_spat.T
    tt = jnp.tile(t_enc.T, (1, _CPT))

    tiles_per_batch = n_chans // _CPT
    cols = _CPT * n_times
    grid_spec = pltpu.PrefetchScalarGridSpec(
        num_scalar_prefetch=1,
        grid=(batch_size, tiles_per_batch),
        in_specs=[
            pl.BlockSpec((SPAT_DIM, NUM_CHANNELS), lambda b, j, idx: (0, 0)),
            pl.BlockSpec((TIME_DIM, cols), lambda b, j, idx: (0, 0)),
        ],
        out_specs=pl.BlockSpec((1, emb_dim, cols), lambda b, j, idx: (b, 0, j)),
    )
    out_t = pl.pallas_call(
        _encode_kernel,
        grid_spec=grid_spec,
        out_shape=jax.ShapeDtypeStruct(
            (batch_size, emb_dim, n_chans_times), jnp.float32
        ),
    )(ch_idxs, wt, tt)
    return out_t.transpose(0, 2, 1)


# speedup vs baseline: 1.0942x; 1.0942x over previous
import math

import jax
import jax.numpy as jnp
from jax.experimental import pallas as pl
from jax.experimental.pallas import tpu as pltpu

SPAT_DIM = 48
TIME_DIM = 48
MAX_N_TIMES = 30000
NUM_CHANNELS = 64

_CPT = 8


def _time_encoding(n_times: int) -> jnp.ndarray:
    position = jnp.arange(n_times, dtype=jnp.float32)[:, None]
    div_term = jnp.exp(
        jnp.arange(0, TIME_DIM, 2, dtype=jnp.float32)
        * (-math.log(MAX_N_TIMES) / TIME_DIM)
    )
    s = jnp.sin(position * div_term)
    c = jnp.cos(position * div_term)
    return jnp.stack([s, c], axis=-1).reshape(n_times, TIME_DIM)


def _encode_kernel(idx_ref, wt_ref, tt_ref, out_ref):
    b = pl.program_id(0)
    j = pl.program_id(1)
    n_times = tt_ref.shape[1] // _CPT
    out_ref[0, pl.ds(SPAT_DIM, TIME_DIM), :] = tt_ref[:, :]
    chan_iota = jax.lax.broadcasted_iota(
        jnp.int32, (SPAT_DIM, NUM_CHANNELS), 1
    )
    wt = wt_ref[:, :]
    for k in range(_CPT):
        cidx = idx_ref[b, j * _CPT + k]
        col = jnp.sum(
            jnp.where(chan_iota == cidx, wt, 0.0), axis=1, keepdims=True
        )
        out_ref[0, pl.ds(0, SPAT_DIM), pl.ds(k * n_times, n_times)] = (
            jnp.broadcast_to(col, (SPAT_DIM, n_times))
        )


def kernel(local_features, ch_idxs, W_spat):
    batch_size, n_chans_times, emb_dim = local_features.shape
    _, n_chans = ch_idxs.shape
    n_times = n_chans_times // n_chans
    t_enc = _time_encoding(n_times)
    wt = W_spat.T
    tt = jnp.tile(t_enc.T, (1, _CPT))

    tiles_per_batch = n_chans // _CPT
    cols = _CPT * n_times
    grid_spec = pltpu.PrefetchScalarGridSpec(
        num_scalar_prefetch=1,
        grid=(batch_size, tiles_per_batch),
        in_specs=[
            pl.BlockSpec((SPAT_DIM, NUM_CHANNELS), lambda b, j, idx: (0, 0)),
            pl.BlockSpec((TIME_DIM, cols), lambda b, j, idx: (0, 0)),
        ],
        out_specs=pl.BlockSpec((1, emb_dim, cols), lambda b, j, idx: (b, 0, j)),
    )
    out_t = pl.pallas_call(
        _encode_kernel,
        grid_spec=grid_spec,
        out_shape=jax.ShapeDtypeStruct(
            (batch_size, emb_dim, n_chans_times), jnp.float32
        ),
    )(ch_idxs, wt, tt)
    return out_t.transpose(0, 2, 1)


# speedup vs baseline: 1.4022x; 1.2814x over previous
import math

import jax
import jax.numpy as jnp
from jax.experimental import pallas as pl
from jax.experimental.pallas import tpu as pltpu

SPAT_DIM = 48
TIME_DIM = 48
MAX_N_TIMES = 30000
NUM_CHANNELS = 64

_CPT = 16


def _time_encoding(n_times: int) -> jnp.ndarray:
    position = jnp.arange(n_times, dtype=jnp.float32)[:, None]
    div_term = jnp.exp(
        jnp.arange(0, TIME_DIM, 2, dtype=jnp.float32)
        * (-math.log(MAX_N_TIMES) / TIME_DIM)
    )
    s = jnp.sin(position * div_term)
    c = jnp.cos(position * div_term)
    return jnp.stack([s, c], axis=-1).reshape(n_times, TIME_DIM)


def _encode_kernel(idx_ref, wt_ref, tt_ref, out_ref):
    b = pl.program_id(0)
    j = pl.program_id(1)
    n_times = tt_ref.shape[1] // _CPT
    out_ref[0, pl.ds(SPAT_DIM, TIME_DIM), :] = tt_ref[:, :]
    chan_iota = jax.lax.broadcasted_iota(
        jnp.int32, (SPAT_DIM, NUM_CHANNELS), 1
    )
    wt = wt_ref[:, :]
    for k in range(_CPT):
        cidx = idx_ref[b, j * _CPT + k]
        col = jnp.sum(
            jnp.where(chan_iota == cidx, wt, 0.0), axis=1, keepdims=True
        )
        out_ref[0, pl.ds(0, SPAT_DIM), pl.ds(k * n_times, n_times)] = (
            jnp.broadcast_to(col, (SPAT_DIM, n_times))
        )


def kernel(local_features, ch_idxs, W_spat):
    batch_size, n_chans_times, emb_dim = local_features.shape
    _, n_chans = ch_idxs.shape
    n_times = n_chans_times // n_chans
    t_enc = _time_encoding(n_times)
    wt = W_spat.T
    tt = jnp.tile(t_enc.T, (1, _CPT))

    tiles_per_batch = n_chans // _CPT
    cols = _CPT * n_times
    grid_spec = pltpu.PrefetchScalarGridSpec(
        num_scalar_prefetch=1,
        grid=(batch_size, tiles_per_batch),
        in_specs=[
            pl.BlockSpec((SPAT_DIM, NUM_CHANNELS), lambda b, j, idx: (0, 0)),
            pl.BlockSpec((TIME_DIM, cols), lambda b, j, idx: (0, 0)),
        ],
        out_specs=pl.BlockSpec((1, emb_dim, cols), lambda b, j, idx: (b, 0, j)),
    )
    out_t = pl.pallas_call(
        _encode_kernel,
        grid_spec=grid_spec,
        out_shape=jax.ShapeDtypeStruct(
            (batch_size, emb_dim, n_chans_times), jnp.float32
        ),
    )(ch_idxs, wt, tt)
    return out_t.transpose(0, 2, 1)
